# baseline (speedup 1.0000x reference)
import jax
import jax.numpy as jnp
from jax import lax
from jax.experimental import pallas as pl
from jax.experimental.pallas import tpu as pltpu
from jax.experimental.pallas import tpu_sc as plsc

N = 10000
D_IN = 128
HID = 128
HEADS = 8
NP = 10112
ROWW = 144
TW16 = 160
EB = 64
LRELU_SLOPE = 0.2
EPS = 1e-16
NEG = -1e30

_PERM = [0] * HID
for _t in range(HID // 32):
    for _u in range(16):
        _PERM[32 * _t + _u] = 32 * _t + 2 * _u
        _PERM[32 * _t + 16 + _u] = 32 * _t + 2 * _u + 1

_mesh = plsc.VectorSubcoreMesh(core_axis_name="c", subcore_axis_name="s")


def _build_sc_edge(n_jobs, passes_per_core, blocks_per_subcore, heads_total,
                   ep, feats_bf16):
    nb = blocks_per_subcore

    def body(taug_ref, ad_ref, src_ref, dst_ref, cvec_ref, out_ref, *scr):
        acc_sh = scr[0]
        H = list(scr[1:4])
        if feats_bf16:
            MSG = list(scr[4:6])
            rest = scr[6:]
        else:
            MSG = H
            rest = scr[4:]
        BR = list(rest[0:3])
        GX = list(rest[3:6])
        GY = list(rest[6:9])
        SX = list(rest[9:12])
        SI = list(rest[12:15])
        DI = list(rest[15:18])
        cbuf = rest[18]
        MI = list(rest[19:22])
        MJ = list(rest[22:25])
        MG = list(rest[25:28])
        MH = list(rest[28:31])
        MS = list(rest[31:34])
        c = lax.axis_index("c")
        s = lax.axis_index("s")
        z16 = jnp.zeros((16,), jnp.float32)

        pltpu.sync_copy(cvec_ref, cbuf)
        cv = cbuf[...]

        rows_per_sub = NP // 16
        row0 = s * rows_per_sub
        zsrc = MSG[0] if feats_bf16 else H[0]

        def issue_idx(slot, base):
            pltpu.async_copy(src_ref.at[pl.ds(base, EB)], SI[slot], MI[slot])
            pltpu.async_copy(dst_ref.at[pl.ds(base, EB)], DI[slot], MJ[slot])

        def wait_idx(slot):
            pltpu.make_async_copy(src_ref.at[pl.ds(0, EB)], SI[slot],
                                  MI[slot]).wait()
            pltpu.make_async_copy(dst_ref.at[pl.ds(0, EB)], DI[slot],
                                  MJ[slot]).wait()

        def mk_gidx(r, off):
            for t in range(EB // 16):
                GX[r][pl.ds(t * 16, 16)] = SI[r][pl.ds(t * 16, 16)] + off
                GY[r][pl.ds(t * 16, 16)] = DI[r][pl.ds(t * 16, 16)] + off
                SX[r][pl.ds(t * 16, 16)] = DI[r][pl.ds(t * 16, 16)]

        def issue_gather(r):
            pltpu.async_copy(taug_ref.at[GX[r]], H[r], MG[r])
            pltpu.async_copy(ad_ref.at[GY[r]], BR[r], MH[r])

        def wait_gather(r):
            pltpu.make_async_copy(taug_ref.at[GX[r]], H[r], MG[r]).wait()
            pltpu.make_async_copy(ad_ref.at[GY[r]], BR[r], MH[r]).wait()

        def sct_src(r, m):
            return MSG[m] if feats_bf16 else H[r]

        def issue_scatter(r, m):
            pltpu.async_copy(sct_src(r, m), acc_sh.at[SX[r]], MS[r], add=True)

        def wait_scatter(r, m):
            pltpu.make_async_copy(sct_src(r, m), acc_sh.at[SX[r]],
                                  MS[r]).wait()

        def scale_block(r, m):
            hq = H[r]
            bq = BR[r]

            if feats_bf16:
                mq = MSG[m]
                himask = jnp.int32(-65536)

                def expand(w):
                    even = lax.bitcast_convert_type(w << 16, jnp.float32)
                    odd = lax.bitcast_convert_type(w & himask, jnp.float32)
                    return even, odd

                def scale(j, _):
                    av, _ = expand(hq[j, pl.ds(HID // 2, 16)])
                    bv = bq[j, pl.ds(0, 16)]
                    al = av + bv
                    al = jnp.where(al >= 0.0, al, al * LRELU_SLOPE)
                    ebc = jnp.exp(al - cv)
                    for t in range(HID // 32):
                        fa, fb = expand(hq[j, pl.ds(t * 16, 16)])
                        mq[j, pl.ds(t * 32, 16)] = fa * ebc
                        mq[j, pl.ds(t * 32 + 16, 16)] = fb * ebc
                    mq[j, pl.ds(HID, 16)] = ebc
                    return 0
            else:
                def scale(j, _):
                    av = hq[j, pl.ds(HID, 16)]
                    bv = bq[j, pl.ds(0, 16)]
                    al = av + bv
                    al = jnp.where(al >= 0.0, al, al * LRELU_SLOPE)
                    ebc = jnp.exp(al - cv)
                    for t in range(HID // 16):
                        hq[j, pl.ds(t * 16, 16)] = (
                            hq[j, pl.ds(t * 16, 16)] * ebc)
                    hq[j, pl.ds(HID, 16)] = ebc
                    return 0
            lax.fori_loop(0, EB, scale, 0, unroll=4)

        def one_pass(p, _):
            if heads_total == 1:
                head = jnp.int32(0)
                job = c
                edge_off = c * (ep // 2)
            else:
                head = 2 * p + c
                job = head
                edge_off = 0
            off = head * NP

            def base_of(b):
                return edge_off + (s * nb + b) * EB

            def zfill(r, _):
                for t in range(ROWW // 16):
                    zsrc[r, pl.ds(t * 16, 16)] = z16
                return 0
            lax.fori_loop(0, EB, zfill, 0)

            def zero_acc(i, _):
                pltpu.sync_copy(zsrc, acc_sh.at[pl.ds(row0 + i * EB, EB)])
                return 0
            lax.fori_loop(0, rows_per_sub // EB, zero_acc, 0)
            rtail = rows_per_sub % EB
            if rtail:
                pltpu.sync_copy(
                    zsrc.at[pl.ds(0, rtail)],
                    acc_sh.at[pl.ds(row0 + (rows_per_sub // EB) * EB, rtail)])
            plsc.subcore_barrier()

            def block_step(b, b6, first_two, has_next, has_next2):
                r = b6 % 3
                m = b6 % 2
                wait_gather(r)
                if has_next:
                    rn = (b6 + 1) % 3
                    if not first_two:
                        wait_scatter(rn, m)
                    wait_idx(rn)
                    mk_gidx(rn, off)
                    issue_gather(rn)
                    if has_next2:
                        issue_idx((b6 + 2) % 3, base_of(b + 2))
                scale_block(r, m)
                issue_scatter(r, m)

            issue_idx(0, base_of(0))
            issue_idx(1, base_of(1))
            wait_idx(0)
            mk_gidx(0, off)
            issue_gather(0)

            for b6 in range(6):
                block_step(b6, b6, b6 < 2, True, True)

            def mgrp(gi, _):
                for b6 in range(6):
                    block_step(gi * 6 + b6, b6, False, True, True)
                return 0
            lax.fori_loop(1, nb // 6 - 1, mgrp, 0)

            for b6 in range(6):
                b = nb - 6 + b6
                block_step(b, b6, False, b + 1 < nb, b + 2 < nb)
            wait_scatter(0, 1)
            wait_scatter(1, 0)
            wait_scatter(2, 1)
            plsc.subcore_barrier()

            def dump(i, _):
                r = row0 + i * EB
                pltpu.sync_copy(acc_sh.at[pl.ds(r, EB)],
                                out_ref.at[job, pl.ds(r, EB)])
                return 0
            lax.fori_loop(0, rows_per_sub // EB, dump, 0)
            if rtail:
                r = row0 + (rows_per_sub // EB) * EB
                pltpu.sync_copy(acc_sh.at[pl.ds(r, rtail)],
                                out_ref.at[job, pl.ds(r, rtail)])
            plsc.subcore_barrier()
            return 0

        lax.fori_loop(0, passes_per_core, one_pass, 0)

    if feats_bf16:
        hbufs = ([pltpu.VMEM((EB, TW16 // 2), jnp.int32)] * 3
                 + [pltpu.VMEM((EB, ROWW), jnp.float32)] * 2)
    else:
        hbufs = [pltpu.VMEM((EB, ROWW), jnp.float32)] * 3

    return pl.kernel(
        body,
        mesh=_mesh,
        compiler_params=pltpu.CompilerParams(use_tc_tiling_on_sc=False),
        out_type=jax.ShapeDtypeStruct((n_jobs, NP, ROWW), jnp.float32),
        scratch_types=(
            [pltpu.VMEM_SHARED((NP, ROWW), jnp.float32)]
            + hbufs
            + [pltpu.VMEM((EB, 16), jnp.float32)] * 3
            + [pltpu.VMEM((EB,), jnp.int32)] * 15
            + [pltpu.VMEM((16,), jnp.float32)]
            + [pltpu.SemaphoreType.DMA] * 15
        ),
    )


def _prep1_body(x_ref, w1_ref, avs_ref, avd_ref, taug_ref, ad_ref):
    x = x_ref[...]
    i = pl.program_id(0)
    rmask = (lax.broadcasted_iota(jnp.int32, (128, HID), 0) + i * 128) < N
    rid32 = lax.broadcasted_iota(jnp.int32, (128, 32), 0) + i * 128
    for k in range(HEADS):
        hk = jnp.dot(x, w1_ref[k], preferred_element_type=jnp.float32)
        hk = jnp.where(rmask, hk, 0.0)
        ak = jnp.sum(hk * avs_ref[k], axis=1)
        bk = jnp.sum(hk * avd_ref[k], axis=1)
        asp = jnp.broadcast_to(ak[:, None], (128, 32))
        asp = jnp.where(rid32 < N, asp, NEG)
        taug_ref[k] = jnp.concatenate(
            [hk.astype(jnp.bfloat16), asp.astype(jnp.bfloat16)], axis=1)
        ad_ref[k] = jnp.broadcast_to(bk[:, None], (128, 16))


def _mid_body(o1_ref, b1_ref, w2_ref, avs2_ref, avd2_ref,
              taug2_ref, ad2_ref):
    acc = jnp.zeros((128, HID), jnp.float32)
    for k in range(HEADS):
        blk = o1_ref[k]
        num = blk[:, :HID]
        den = blk[:, HID:HID + 1]
        h1k = num / (den + EPS) + b1_ref[k]
        h1k = jnp.where(h1k > 0.0, h1k, jnp.exp(h1k) - 1.0)
        acc = acc + jnp.dot(h1k, w2_ref[k], preferred_element_type=jnp.float32)
    i = pl.program_id(0)
    rid = lax.broadcasted_iota(jnp.int32, (128, 16), 0) + i * 128
    a2 = jnp.sum(acc * avs2_ref[...], axis=1)
    b2 = jnp.sum(acc * avd2_ref[...], axis=1)
    asp = jnp.broadcast_to(a2[:, None], (128, 16))
    asp = jnp.where(rid < N, asp, NEG)
    taug2_ref[...] = jnp.concatenate([acc, asp], axis=1)
    ad2_ref[...] = jnp.broadcast_to(b2[:, None], (128, 16))


def _fin_body(o2_ref, b2_ref, out_ref):
    blk0 = o2_ref[0]
    blk1 = o2_ref[1]
    num = blk0[:, :HID] + blk1[:, :HID]
    den = blk0[:, HID:HID + 1] + blk1[:, HID:HID + 1]
    out_ref[...] = num / (den + EPS) + b2_ref[...]


def _leaky_relu(x):
    return jnp.where(x >= 0.0, x, x * LRELU_SLOPE)


def kernel(x, edge_index, W1, att_src1, att_dst1, bias1,
           W2, att_src2, att_dst2, bias2):
    n = x.shape[0]
    e_real = edge_index.shape[1] + n
    ep = ((e_real + 12287) // 12288) * 12288
    n_pad = ep - e_real

    loop = jnp.arange(n, dtype=jnp.int32)
    src = jnp.concatenate([edge_index[0].astype(jnp.int32), loop,
                           jnp.full((n_pad,), NP - 1, jnp.int32)])
    dst = jnp.concatenate([edge_index[1].astype(jnp.int32), loop,
                           jnp.zeros((n_pad,), jnp.int32)])

    w1r = W1.reshape(D_IN, HEADS, HID).transpose(1, 0, 2)
    avs1 = att_src1.reshape(HEADS, HID)
    avd1 = att_dst1.reshape(HEADS, HID)
    perm = jnp.array(_PERM, dtype=jnp.int32)
    w2r = W2.reshape(HEADS, HID, HID)[:, perm, :]
    b1p = bias1.reshape(HEADS, HID)[:, perm]
    avs2 = att_src2.reshape(1, HID)
    avd2 = att_dst2.reshape(1, HID)
    b2r = bias2.reshape(1, HID)

    grid = NP // 128

    taug1, ad_t = pl.pallas_call(
        _prep1_body,
        grid=(grid,),
        in_specs=[
            pl.BlockSpec((128, D_IN), lambda i: (i, 0)),
            pl.BlockSpec((HEADS, D_IN, HID), lambda i: (0, 0, 0)),
            pl.BlockSpec((HEADS, HID), lambda i: (0, 0)),
            pl.BlockSpec((HEADS, HID), lambda i: (0, 0)),
        ],
        out_specs=[
            pl.BlockSpec((HEADS, 128, TW16), lambda i: (0, i, 0)),
            pl.BlockSpec((HEADS, 128, 16), lambda i: (0, i, 0)),
        ],
        out_shape=[
            jax.ShapeDtypeStruct((HEADS, NP, TW16), jnp.bfloat16),
            jax.ShapeDtypeStruct((HEADS, NP, 16), jnp.float32),
        ],
    )(x, w1r, avs1, avd1)

    c1 = _leaky_relu(jnp.max(taug1[:, :, HID].astype(jnp.float32))
                     + jnp.max(ad_t[:, :, 0]))
    cvec1 = jnp.broadcast_to(c1, (16,)).astype(jnp.float32)

    sc_l1 = _build_sc_edge(
        n_jobs=HEADS, passes_per_core=HEADS // 2,
        blocks_per_subcore=ep // (16 * EB), heads_total=HEADS, ep=ep,
        feats_bf16=True)
    taug1_i32 = jax.lax.bitcast_convert_type(
        taug1.reshape(HEADS * NP, TW16 // 2, 2), jnp.int32)
    out1 = sc_l1(taug1_i32, ad_t.reshape(HEADS * NP, 16),
                 src, dst, cvec1)

    taug2, ad2_t = pl.pallas_call(
        _mid_body,
        grid=(grid,),
        in_specs=[
            pl.BlockSpec((HEADS, 128, ROWW), lambda i: (0, i, 0)),
            pl.BlockSpec((HEADS, HID), lambda i: (0, 0)),
            pl.BlockSpec((HEADS, HID, HID), lambda i: (0, 0, 0)),
            pl.BlockSpec((1, HID), lambda i: (0, 0)),
            pl.BlockSpec((1, HID), lambda i: (0, 0)),
        ],
        out_specs=[
            pl.BlockSpec((128, ROWW), lambda i: (i, 0)),
            pl.BlockSpec((128, 16), lambda i: (i, 0)),
        ],
        out_shape=[
            jax.ShapeDtypeStruct((NP, ROWW), jnp.float32),
            jax.ShapeDtypeStruct((NP, 16), jnp.float32),
        ],
    )(out1, b1p, w2r, avs2, avd2)

    c2 = _leaky_relu(jnp.max(taug2[:, HID]) + jnp.max(ad2_t[:, 0]))
    cvec2 = jnp.broadcast_to(c2, (16,)).astype(jnp.float32)

    sc_l2 = _build_sc_edge(
        n_jobs=2, passes_per_core=1,
        blocks_per_subcore=ep // (2 * 16 * EB), heads_total=1, ep=ep,
        feats_bf16=False)
    out2 = sc_l2(taug2, ad2_t, src, dst, cvec2)

    out = pl.pallas_call(
        _fin_body,
        grid=(grid,),
        in_specs=[
            pl.BlockSpec((2, 128, ROWW), lambda i: (0, i, 0)),
            pl.BlockSpec((1, HID), lambda i: (0, 0)),
        ],
        out_specs=pl.BlockSpec((128, HID), lambda i: (i, 0)),
        out_shape=jax.ShapeDtypeStruct((n, HID), jnp.float32),
    )(out2, b2r)

    return out


# speedup vs baseline: 1.7270x; 1.7270x over previous
import jax
import jax.numpy as jnp
from jax import lax
from jax.experimental import pallas as pl
from jax.experimental.pallas import tpu as pltpu
from jax.experimental.pallas import tpu_sc as plsc

N = 10000
D_IN = 128
HID = 128
HEADS = 8
NP = 10112
ROWW = 144
EB = 80
LRELU_SLOPE = 0.2
EPS = 1e-16
NEG = -1e30

_mesh = plsc.VectorSubcoreMesh(core_axis_name="c", subcore_axis_name="s")


def _build_sc_edge(n_jobs, passes_per_core, blocks_per_subcore, heads_total,
                   ep):
    nb = blocks_per_subcore

    def body(taug_ref, ad_ref, src_ref, dst_ref, cvec_ref, out_ref,
             acc_sh,
             h0, h1, h2, br0, br1, br2, gx0, gx1, gx2, gy0, gy1, gy2,
             sx0, sx1, sx2, si0, si1, si2, di0, di1, di2, cbuf,
             mi0, mi1, mi2, mj0, mj1, mj2,
             mg0, mg1, mg2, mh0, mh1, mh2, ms0, ms1, ms2):
        c = lax.axis_index("c")
        s = lax.axis_index("s")
        z16 = jnp.zeros((16,), jnp.float32)
        H = [h0, h1, h2]
        BR = [br0, br1, br2]
        GX = [gx0, gx1, gx2]
        GY = [gy0, gy1, gy2]
        SX = [sx0, sx1, sx2]
        SI = [si0, si1, si2]
        DI = [di0, di1, di2]
        MI = [mi0, mi1, mi2]
        MJ = [mj0, mj1, mj2]
        MG = [mg0, mg1, mg2]
        MH = [mh0, mh1, mh2]
        MS = [ms0, ms1, ms2]

        pltpu.sync_copy(cvec_ref, cbuf)
        cv = cbuf[...]

        rows_per_sub = NP // 16
        rchunk = rows_per_sub // 8
        row0 = s * rows_per_sub

        def issue_idx(slot, base):
            pltpu.async_copy(src_ref.at[pl.ds(base, EB)], SI[slot], MI[slot])
            pltpu.async_copy(dst_ref.at[pl.ds(base, EB)], DI[slot], MJ[slot])

        def wait_idx(slot):
            pltpu.make_async_copy(src_ref.at[pl.ds(0, EB)], SI[slot],
                                  MI[slot]).wait()
            pltpu.make_async_copy(dst_ref.at[pl.ds(0, EB)], DI[slot],
                                  MJ[slot]).wait()

        def mk_gidx(r, off):
            for t in range(EB // 16):
                GX[r][pl.ds(t * 16, 16)] = SI[r][pl.ds(t * 16, 16)] + off
                GY[r][pl.ds(t * 16, 16)] = DI[r][pl.ds(t * 16, 16)] + off
                SX[r][pl.ds(t * 16, 16)] = DI[r][pl.ds(t * 16, 16)]

        def issue_gather(r):
            pltpu.async_copy(taug_ref.at[GX[r]], H[r], MG[r])
            pltpu.async_copy(ad_ref.at[GY[r]], BR[r], MH[r])

        def wait_gather(r):
            pltpu.make_async_copy(taug_ref.at[GX[r]], H[r], MG[r]).wait()
            pltpu.make_async_copy(ad_ref.at[GY[r]], BR[r], MH[r]).wait()

        def issue_scatter(r):
            pltpu.async_copy(H[r], acc_sh.at[SX[r]], MS[r], add=True)

        def wait_scatter(r):
            pltpu.make_async_copy(H[r], acc_sh.at[SX[r]], MS[r]).wait()

        def scale_block(r):
            hq = H[r]
            bq = BR[r]

            def scale(j, _):
                av = hq[j, pl.ds(HID, 16)]
                bv = bq[j, pl.ds(0, 16)]
                al = av + bv
                al = jnp.where(al >= 0.0, al, al * LRELU_SLOPE)
                ebc = jnp.exp(al - cv)
                for t in range(HID // 16):
                    hq[j, pl.ds(t * 16, 16)] = hq[j, pl.ds(t * 16, 16)] * ebc
                hq[j, pl.ds(HID, 16)] = ebc
                return 0
            lax.fori_loop(0, EB, scale, 0, unroll=4)

        def one_pass(p, _):
            if heads_total == 1:
                head = jnp.int32(0)
                job = c
                edge_off = c * (ep // 2)
            else:
                head = 2 * p + c
                job = head
                edge_off = 0
            off = head * NP

            def base_of(b):
                return edge_off + (s * nb + b) * EB

            def zfill(r, _):
                for t in range(ROWW // 16):
                    h0[r, pl.ds(t * 16, 16)] = z16
                return 0
            lax.fori_loop(0, rchunk, zfill, 0)

            def zero_acc(i, _):
                pltpu.sync_copy(h0.at[pl.ds(0, rchunk)],
                                acc_sh.at[pl.ds(row0 + i * rchunk, rchunk)])
                return 0
            lax.fori_loop(0, 8, zero_acc, 0)
            plsc.subcore_barrier()

            def block_step(b, b3, first_two, has_next, has_next2):
                r = b3 % 3
                wait_gather(r)
                if has_next:
                    rn = (b3 + 1) % 3
                    if not first_two:
                        wait_scatter(rn)
                    wait_idx(rn)
                    mk_gidx(rn, off)
                    issue_gather(rn)
                    if has_next2:
                        issue_idx((b3 + 2) % 3, base_of(b + 2))
                scale_block(r)
                issue_scatter(r)

            issue_idx(0, base_of(0))
            issue_idx(1, base_of(1))
            wait_idx(0)
            mk_gidx(0, off)
            issue_gather(0)

            for b3 in range(3):
                block_step(b3, b3, b3 < 2, True, True)

            def mgrp(gi, _):
                for b3 in range(3):
                    block_step(gi * 3 + b3, b3, False, True, True)
                return 0
            lax.fori_loop(1, nb // 3 - 1, mgrp, 0)

            for b3 in range(3):
                b = nb - 3 + b3
                block_step(b, b3, False, b + 1 < nb, b + 2 < nb)
            wait_scatter(0)
            wait_scatter(1)
            wait_scatter(2)
            plsc.subcore_barrier()

            def dump(i, _):
                r = row0 + i * rchunk
                pltpu.sync_copy(acc_sh.at[pl.ds(r, rchunk)],
                                out_ref.at[job, pl.ds(r, rchunk)])
                return 0
            lax.fori_loop(0, 8, dump, 0)
            plsc.subcore_barrier()
            return 0

        lax.fori_loop(0, passes_per_core, one_pass, 0)

    return pl.kernel(
        body,
        mesh=_mesh,
        compiler_params=pltpu.CompilerParams(use_tc_tiling_on_sc=False),
        out_type=jax.ShapeDtypeStruct((n_jobs, NP, ROWW), jnp.float32),
        scratch_types=(
            [pltpu.VMEM_SHARED((NP, ROWW), jnp.float32)]
            + [pltpu.VMEM((EB, ROWW), jnp.float32)] * 3
            + [pltpu.VMEM((EB, 16), jnp.float32)] * 3
            + [pltpu.VMEM((EB,), jnp.int32)] * 15
            + [pltpu.VMEM((16,), jnp.float32)]
            + [pltpu.SemaphoreType.DMA] * 15
        ),
    )


def _prep1_body(x_ref, w1_ref, avs_ref, avd_ref, taug_ref, ad_ref):
    x = x_ref[...]
    i = pl.program_id(0)
    rid = lax.broadcasted_iota(jnp.int32, (128, 16), 0) + i * 128
    rmask = (lax.broadcasted_iota(jnp.int32, (128, HID), 0) + i * 128) < N
    for k in range(HEADS):
        hk = jnp.dot(x, w1_ref[k], preferred_element_type=jnp.float32)
        hk = jnp.where(rmask, hk, 0.0)
        ak = jnp.sum(hk * avs_ref[k], axis=1)
        bk = jnp.sum(hk * avd_ref[k], axis=1)
        asp = jnp.broadcast_to(ak[:, None], (128, 16))
        asp = jnp.where(rid < N, asp, NEG)
        taug_ref[k] = jnp.concatenate([hk, asp], axis=1)
        ad_ref[k] = jnp.broadcast_to(bk[:, None], (128, 16))


def _mid_body(o1_ref, b1_ref, w2_ref, avs2_ref, avd2_ref,
              taug2_ref, ad2_ref):
    acc = jnp.zeros((128, HID), jnp.float32)
    for k in range(HEADS):
        blk = o1_ref[k]
        num = blk[:, :HID]
        den = blk[:, HID:HID + 1]
        h1k = num / (den + EPS) + b1_ref[k]
        h1k = jnp.where(h1k > 0.0, h1k, jnp.exp(h1k) - 1.0)
        acc = acc + jnp.dot(h1k, w2_ref[k], preferred_element_type=jnp.float32)
    i = pl.program_id(0)
    rid = lax.broadcasted_iota(jnp.int32, (128, 16), 0) + i * 128
    a2 = jnp.sum(acc * avs2_ref[...], axis=1)
    b2 = jnp.sum(acc * avd2_ref[...], axis=1)
    asp = jnp.broadcast_to(a2[:, None], (128, 16))
    asp = jnp.where(rid < N, asp, NEG)
    taug2_ref[...] = jnp.concatenate([acc, asp], axis=1)
    ad2_ref[...] = jnp.broadcast_to(b2[:, None], (128, 16))


def _fin_body(o2_ref, b2_ref, out_ref):
    blk0 = o2_ref[0]
    blk1 = o2_ref[1]
    num = blk0[:, :HID] + blk1[:, :HID]
    den = blk0[:, HID:HID + 1] + blk1[:, HID:HID + 1]
    out_ref[...] = num / (den + EPS) + b2_ref[...]


def _leaky_relu(x):
    return jnp.where(x >= 0.0, x, x * LRELU_SLOPE)


def kernel(x, edge_index, W1, att_src1, att_dst1, bias1,
           W2, att_src2, att_dst2, bias2):
    n = x.shape[0]
    e_real = edge_index.shape[1] + n
    ep = ((e_real + 7679) // 7680) * 7680
    n_pad = ep - e_real

    loop = jnp.arange(n, dtype=jnp.int32)
    src = jnp.concatenate([edge_index[0].astype(jnp.int32), loop,
                           jnp.full((n_pad,), NP - 1, jnp.int32)])
    dst = jnp.concatenate([edge_index[1].astype(jnp.int32), loop,
                           jnp.zeros((n_pad,), jnp.int32)])

    w1r = W1.reshape(D_IN, HEADS, HID).transpose(1, 0, 2)
    avs1 = att_src1.reshape(HEADS, HID)
    avd1 = att_dst1.reshape(HEADS, HID)
    w2r = W2.reshape(HEADS, HID, HID)
    avs2 = att_src2.reshape(1, HID)
    avd2 = att_dst2.reshape(1, HID)
    b1r = bias1.reshape(HEADS, HID)
    b2r = bias2.reshape(1, HID)

    grid = NP // 128

    taug1, ad_t = pl.pallas_call(
        _prep1_body,
        grid=(grid,),
        in_specs=[
            pl.BlockSpec((128, D_IN), lambda i: (i, 0)),
            pl.BlockSpec((HEADS, D_IN, HID), lambda i: (0, 0, 0)),
            pl.BlockSpec((HEADS, HID), lambda i: (0, 0)),
            pl.BlockSpec((HEADS, HID), lambda i: (0, 0)),
        ],
        out_specs=[
            pl.BlockSpec((HEADS, 128, ROWW), lambda i: (0, i, 0)),
            pl.BlockSpec((HEADS, 128, 16), lambda i: (0, i, 0)),
        ],
        out_shape=[
            jax.ShapeDtypeStruct((HEADS, NP, ROWW), jnp.float32),
            jax.ShapeDtypeStruct((HEADS, NP, 16), jnp.float32),
        ],
    )(x, w1r, avs1, avd1)

    c1 = _leaky_relu(jnp.max(taug1[:, :, HID]) + jnp.max(ad_t[:, :, 0]))
    cvec1 = jnp.broadcast_to(c1, (16,)).astype(jnp.float32)

    sc_l1 = _build_sc_edge(
        n_jobs=HEADS, passes_per_core=HEADS // 2,
        blocks_per_subcore=ep // (16 * EB), heads_total=HEADS, ep=ep)
    out1 = sc_l1(taug1.reshape(HEADS * NP, ROWW),
                 ad_t.reshape(HEADS * NP, 16),
                 src, dst, cvec1)

    taug2, ad2_t = pl.pallas_call(
        _mid_body,
        grid=(grid,),
        in_specs=[
            pl.BlockSpec((HEADS, 128, ROWW), lambda i: (0, i, 0)),
            pl.BlockSpec((HEADS, HID), lambda i: (0, 0)),
            pl.BlockSpec((HEADS, HID, HID), lambda i: (0, 0, 0)),
            pl.BlockSpec((1, HID), lambda i: (0, 0)),
            pl.BlockSpec((1, HID), lambda i: (0, 0)),
        ],
        out_specs=[
            pl.BlockSpec((128, ROWW), lambda i: (i, 0)),
            pl.BlockSpec((128, 16), lambda i: (i, 0)),
        ],
        out_shape=[
            jax.ShapeDtypeStruct((NP, ROWW), jnp.float32),
            jax.ShapeDtypeStruct((NP, 16), jnp.float32),
        ],
    )(out1, b1r, w2r, avs2, avd2)

    c2 = _leaky_relu(jnp.max(taug2[:, HID]) + jnp.max(ad2_t[:, 0]))
    cvec2 = jnp.broadcast_to(c2, (16,)).astype(jnp.float32)

    sc_l2 = _build_sc_edge(
        n_jobs=2, passes_per_core=1,
        blocks_per_subcore=ep // (2 * 16 * EB), heads_total=1, ep=ep)
    out2 = sc_l2(taug2, ad2_t, src, dst, cvec2)

    out = pl.pallas_call(
        _fin_body,
        grid=(grid,),
        in_specs=[
            pl.BlockSpec((2, 128, ROWW), lambda i: (0, i, 0)),
            pl.BlockSpec((1, HID), lambda i: (0, 0)),
        ],
        out_specs=pl.BlockSpec((128, HID), lambda i: (i, 0)),
        out_shape=jax.ShapeDtypeStruct((n, HID), jnp.float32),
    )(out2, b2r)

    return out
